# baseline (speedup 1.0000x reference)
import functools

import jax
import jax.numpy as jnp
from jax import lax
from jax.experimental import pallas as pl
from jax.experimental.pallas import tpu as pltpu
from jax.experimental.pallas import tpu_sc as plsc

NUM_CORES = 2
NUM_SUBCORES = 16
NUM_WORKERS = NUM_CORES * NUM_SUBCORES

BLOCK_R = 128
CHUNK_B = 8
CHUNK_W = CHUNK_B * 512
CHUNK_R = CHUNK_B * 128


def _compute_groups(pbuf, tbuf, obuf, n_groups):
  def _group(i, _):
    base = ((i >> 3) << 9) + ((i & 7) << 4)
    px = pbuf[pl.ds(base, 16)]
    py = pbuf[pl.ds(base + 128, 16)]
    pw = pbuf[pl.ds(base + 256, 16)]
    ph = pbuf[pl.ds(base + 384, 16)]
    tx = tbuf[pl.ds(base, 16)]
    ty = tbuf[pl.ds(base + 128, 16)]
    tw = tbuf[pl.ds(base + 256, 16)]
    th = tbuf[pl.ds(base + 384, 16)]
    dx = jnp.minimum((pw + tw) * 0.5 - jnp.abs(px - tx), jnp.minimum(pw, tw))
    dy = jnp.minimum((ph + th) * 0.5 - jnp.abs(py - ty), jnp.minimum(ph, th))
    area_i = jnp.maximum(dx, 0.0) * jnp.maximum(dy, 0.0)
    area_u = pw * ph + tw * th - area_i
    iou = area_i / area_u
    obuf[pl.ds(i * 16, 16)] = 1.0 - iou * iou
    return 0

  lax.fori_loop(0, n_groups, _group, 0, unroll=4)


@functools.lru_cache(maxsize=None)
def _build(n_rows):
  assert n_rows % BLOCK_R == 0
  n_blocks = n_rows // BLOCK_R
  full = n_blocks // (NUM_WORKERS * CHUNK_B)
  per_w = full * CHUNK_B
  rem = n_blocks - per_w * NUM_WORKERS
  assert rem < NUM_WORKERS

  mesh = plsc.VectorSubcoreMesh(
      core_axis_name="c", subcore_axis_name="s",
      num_cores=NUM_CORES, num_subcores=NUM_SUBCORES)

  def body(pred_hbm, tgt_hbm, out_hbm,
           p0, p1, t0, t1, o0, o1, si0, si1, so0, so1):
    w = lax.axis_index("s") * NUM_CORES + lax.axis_index("c")
    b0 = w * per_w

    pbufs, tbufs, obufs = (p0, p1), (t0, t1), (o0, o1)
    sis, sos = (si0, si1), (so0, so1)

    def start_in(t, s):
      off = (b0 + t * CHUNK_B) * 512
      pltpu.async_copy(pred_hbm.at[pl.ds(off, CHUNK_W)], pbufs[s], sis[s])
      pltpu.async_copy(tgt_hbm.at[pl.ds(off, CHUNK_W)], tbufs[s], sis[s])

    def do_chunk(t, s):
      @pl.when(t + 1 < full)
      def _():
        start_in(t + 1, 1 - s)
      off = (b0 + t * CHUNK_B) * 512
      pltpu.make_async_copy(
          pred_hbm.at[pl.ds(off, CHUNK_W)], pbufs[s], sis[s]).wait()
      pltpu.make_async_copy(
          tgt_hbm.at[pl.ds(off, CHUNK_W)], tbufs[s], sis[s]).wait()
      @pl.when(t >= 2)
      def _():
        off2 = (b0 + (t - 2) * CHUNK_B) * 128
        pltpu.make_async_copy(
            obufs[s], out_hbm.at[pl.ds(off2, CHUNK_R)], sos[s]).wait()
      _compute_groups(pbufs[s], tbufs[s], obufs[s], CHUNK_R // 16)
      off3 = (b0 + t * CHUNK_B) * 128
      pltpu.async_copy(obufs[s], out_hbm.at[pl.ds(off3, CHUNK_R)], sos[s])

    if full > 0:
      start_in(0, 0)

      def loop_body(t, _):
        @pl.when(t % 2 == 0)
        def _():
          do_chunk(t, 0)

        @pl.when(t % 2 == 1)
        def _():
          do_chunk(t, 1)
        return 0

      lax.fori_loop(0, full, loop_body, 0)
      for tt in range(max(full - 2, 0), full):
        s = tt % 2
        off = (b0 + tt * CHUNK_B) * 128
        pltpu.make_async_copy(
            obufs[s], out_hbm.at[pl.ds(off, CHUNK_R)], sos[s]).wait()

    if rem:
      @pl.when(w < rem)
      def _():
        blk = per_w * NUM_WORKERS + w
        pltpu.sync_copy(pred_hbm.at[pl.ds(blk * 512, 512)], p0.at[pl.ds(0, 512)])
        pltpu.sync_copy(tgt_hbm.at[pl.ds(blk * 512, 512)], t0.at[pl.ds(0, 512)])
        _compute_groups(p0, t0, o0, BLOCK_R // 16)
        pltpu.sync_copy(o0.at[pl.ds(0, 128)], out_hbm.at[pl.ds(blk * 128, 128)])

  f32 = jnp.float32
  return pl.kernel(
      body,
      out_type=jax.ShapeDtypeStruct((n_rows,), f32),
      mesh=mesh,
      compiler_params=pltpu.CompilerParams(needs_layout_passes=False),
      scratch_types=[
          pltpu.VMEM((CHUNK_W,), f32), pltpu.VMEM((CHUNK_W,), f32),
          pltpu.VMEM((CHUNK_W,), f32), pltpu.VMEM((CHUNK_W,), f32),
          pltpu.VMEM((CHUNK_R,), f32), pltpu.VMEM((CHUNK_R,), f32),
          pltpu.SemaphoreType.DMA, pltpu.SemaphoreType.DMA,
          pltpu.SemaphoreType.DMA, pltpu.SemaphoreType.DMA,
      ],
  )


def _to_physical(x, n_rows):
  return x.reshape(n_rows // BLOCK_R, BLOCK_R, 4).transpose(0, 2, 1).reshape(-1)


def kernel(pred, target):
  pred = pred.reshape(-1, 4)
  target = target.reshape(-1, 4)
  n_rows = pred.shape[0]
  fn = _build(n_rows)
  return fn(_to_physical(pred, n_rows), _to_physical(target, n_rows))
